# baseline (speedup 1.0000x reference)
import jax
import jax.numpy as jnp
from jax.experimental import pallas as pl
from jax.experimental.pallas import tpu as pltpu

_PRE = 2000
_PAD = 2048
_ROWS = _PAD // 128
_POST = 100
_IOU_THR = 0.5
_SCORE_THR = 0.05


def _nms_kernel(x1_ref, y1_ref, x2_ref, y2_ref, s_ref, idx_ref,
                ox1_ref, oy1_ref, ox2_ref, oy2_ref, osc_ref, oidx_ref,
                keep_ref):
    scores = s_ref[...]
    x1 = x1_ref[...]
    y1 = y1_ref[...]
    x2 = x2_ref[...]
    y2 = y2_ref[...]
    area = (x2 - x1) * (y2 - y1)
    flat = (jax.lax.broadcasted_iota(jnp.int32, (_ROWS, 128), 0) * 128
            + jax.lax.broadcasted_iota(jnp.int32, (_ROWS, 128), 1))
    lane = jax.lax.broadcasted_iota(jnp.int32, (1, 128), 1)

    keep_ref[...] = (scores > 0.0).astype(jnp.float32)
    ox1_ref[...] = jnp.zeros((1, 128), jnp.float32)
    oy1_ref[...] = jnp.zeros((1, 128), jnp.float32)
    ox2_ref[...] = jnp.zeros((1, 128), jnp.float32)
    oy2_ref[...] = jnp.zeros((1, 128), jnp.float32)
    osc_ref[...] = jnp.full((1, 128), -1.0, jnp.float32)
    oidx_ref[...] = jnp.zeros((1, 128), jnp.int32)

    def pivot_scalars(r, oh, ohf):
        x1i = jnp.sum(x1_ref[pl.ds(r, 1), :] * ohf)
        y1i = jnp.sum(y1_ref[pl.ds(r, 1), :] * ohf)
        x2i = jnp.sum(x2_ref[pl.ds(r, 1), :] * ohf)
        y2i = jnp.sum(y2_ref[pl.ds(r, 1), :] * ohf)
        sci = jnp.sum(s_ref[pl.ds(r, 1), :] * ohf)
        idxi = jnp.sum(idx_ref[pl.ds(r, 1), :] * oh.astype(jnp.int32))
        return x1i, y1i, x2i, y2i, sci, idxi

    def emit(cnt, x1i, y1i, x2i, y2i, sci, idxi):
        oh = lane == cnt
        ox1_ref[...] = jnp.where(oh, x1i, ox1_ref[...])
        oy1_ref[...] = jnp.where(oh, y1i, oy1_ref[...])
        ox2_ref[...] = jnp.where(oh, x2i, ox2_ref[...])
        oy2_ref[...] = jnp.where(oh, y2i, oy2_ref[...])
        osc_ref[...] = jnp.where(oh, sci, osc_ref[...])
        oidx_ref[...] = jnp.where(oh, idxi, oidx_ref[...])

    def cond(state):
        i, cnt = state
        return (i < _PAD) & (cnt < _POST)

    def body(state):
        i, cnt = state
        r = i // 128
        c = i % 128
        oh = lane == c
        ohf = oh.astype(jnp.float32)
        x1i, y1i, x2i, y2i, sci, idxi = pivot_scalars(r, oh, ohf)
        emit(cnt, x1i, y1i, x2i, y2i, sci, idxi)
        area_i = (x2i - x1i) * (y2i - y1i)
        xx1 = jnp.maximum(x1i, x1)
        yy1 = jnp.maximum(y1i, y1)
        xx2 = jnp.minimum(x2i, x2)
        yy2 = jnp.minimum(y2i, y2)
        inter = (jnp.clip(xx2 - xx1, 0.0, None)
                 * jnp.clip(yy2 - yy1, 0.0, None))
        union = area_i + area - inter
        iou = inter / jnp.maximum(union, 1e-8)
        sup = (iou > _IOU_THR) & (flat > i)
        keep_new = jnp.where(sup, 0.0, keep_ref[...])
        keep_ref[...] = keep_new
        nxt = jnp.min(jnp.where((keep_new > 0.0) & (flat > i), flat, _PAD))
        return nxt, cnt + 1

    valid = keep_ref[...] > 0.0
    next0 = jnp.min(jnp.where(valid, flat, _PAD))
    _, cnt_fin = jax.lax.while_loop(cond, body, (next0, 0))

    def pad_cond(state):
        p, cnt = state
        return (p < _PRE) & (cnt < _POST)

    def pad_body(state):
        p, cnt = state
        r = p // 128
        c = p % 128
        oh = lane == c
        ohf = oh.astype(jnp.float32)
        dead = jnp.sum(keep_ref[pl.ds(r, 1), :] * ohf) == 0.0

        @pl.when(dead)
        def _():
            x1i, y1i, x2i, y2i, _sci, idxi = pivot_scalars(r, oh, ohf)
            emit(cnt, x1i, y1i, x2i, y2i, -1.0, idxi)

        return p + 1, cnt + dead.astype(jnp.int32)

    jax.lax.while_loop(pad_cond, pad_body, (0, cnt_fin))


def kernel(boxes, scores):
    masked = jnp.where(scores >= _SCORE_THR, scores, -1.0)
    top_scores, idx = jax.lax.top_k(masked, _PRE)
    top_boxes = jnp.take(boxes, idx, axis=0)

    pad = _PAD - _PRE
    sp = jnp.pad(top_scores, (0, pad), constant_values=-1.0).reshape(_ROWS, 128)
    ip = jnp.pad(idx, (0, pad)).reshape(_ROWS, 128)
    bp = jnp.pad(top_boxes, ((0, pad), (0, 0)))
    x1 = bp[:, 0].reshape(_ROWS, 128)
    y1 = bp[:, 1].reshape(_ROWS, 128)
    x2 = bp[:, 2].reshape(_ROWS, 128)
    y2 = bp[:, 3].reshape(_ROWS, 128)

    out_shapes = [jax.ShapeDtypeStruct((1, 128), jnp.float32)] * 5 + [
        jax.ShapeDtypeStruct((1, 128), jnp.int32)
    ]
    ox1, oy1, ox2, oy2, osc, oidx = pl.pallas_call(
        _nms_kernel,
        out_shape=out_shapes,
        scratch_shapes=[
            pltpu.VMEM((_ROWS, 128), jnp.float32),
        ],
    )(x1, y1, x2, y2, sp, ip)

    sel_boxes = jnp.stack(
        [ox1[0, :_POST], oy1[0, :_POST], ox2[0, :_POST], oy2[0, :_POST]],
        axis=1,
    )
    return sel_boxes, osc[0, :_POST], oidx[0, :_POST]


# speedup vs baseline: 1.1217x; 1.1217x over previous
import jax
import jax.numpy as jnp
from jax.experimental import pallas as pl
from jax.experimental.pallas import tpu as pltpu

_PRE = 2000
_PAD = 2048
_ROWS = _PAD // 128
_POST = 100
_IOU_THR = 0.5
_SCORE_THR = 0.05


def _nms_kernel(x1_ref, y1_ref, x2_ref, y2_ref, s_ref,
                sx1_ref, sy1_ref, sx2_ref, sy2_ref, ss_ref, sidx_ref,
                ox1_ref, oy1_ref, ox2_ref, oy2_ref, osc_ref, oidx_ref,
                keep_ref):
    scores = s_ref[...]
    x1 = x1_ref[...]
    y1 = y1_ref[...]
    x2 = x2_ref[...]
    y2 = y2_ref[...]
    area = (x2 - x1) * (y2 - y1)
    flat = (jax.lax.broadcasted_iota(jnp.int32, (_ROWS, 128), 0) * 128
            + jax.lax.broadcasted_iota(jnp.int32, (_ROWS, 128), 1))
    lane = jax.lax.broadcasted_iota(jnp.int32, (1, 128), 1)

    valid = scores > 0.0
    keep_ref[...] = valid.astype(jnp.float32)
    ox1_ref[...] = jnp.zeros((1, 128), jnp.float32)
    oy1_ref[...] = jnp.zeros((1, 128), jnp.float32)
    ox2_ref[...] = jnp.zeros((1, 128), jnp.float32)
    oy2_ref[...] = jnp.zeros((1, 128), jnp.float32)
    osc_ref[...] = jnp.full((1, 128), -1.0, jnp.float32)
    oidx_ref[...] = jnp.zeros((1, 128), jnp.int32)

    def emit(cnt, i, sci):
        oh = lane == cnt
        ox1_ref[...] = jnp.where(oh, sx1_ref[i], ox1_ref[...])
        oy1_ref[...] = jnp.where(oh, sy1_ref[i], oy1_ref[...])
        ox2_ref[...] = jnp.where(oh, sx2_ref[i], ox2_ref[...])
        oy2_ref[...] = jnp.where(oh, sy2_ref[i], oy2_ref[...])
        osc_ref[...] = jnp.where(oh, sci, osc_ref[...])
        oidx_ref[...] = jnp.where(oh, sidx_ref[i], oidx_ref[...])

    def cond(state):
        i, cnt = state
        return (i < _PAD) & (cnt < _POST)

    def body(state):
        i, cnt = state
        emit(cnt, i, ss_ref[i])
        x1i = sx1_ref[i]
        y1i = sy1_ref[i]
        x2i = sx2_ref[i]
        y2i = sy2_ref[i]
        area_i = (x2i - x1i) * (y2i - y1i)
        xx1 = jnp.maximum(x1i, x1)
        yy1 = jnp.maximum(y1i, y1)
        xx2 = jnp.minimum(x2i, x2)
        yy2 = jnp.minimum(y2i, y2)
        inter = (jnp.clip(xx2 - xx1, 0.0, None)
                 * jnp.clip(yy2 - yy1, 0.0, None))
        union = area_i + area - inter
        iou = inter / jnp.maximum(union, 1e-8)
        sup = (iou > _IOU_THR) & (flat > i)
        keep_new = jnp.where(sup, 0.0, keep_ref[...])
        keep_ref[...] = keep_new
        nxt = jnp.min(jnp.where((keep_new > 0.0) & (flat > i), flat, _PAD))
        return nxt, cnt + 1

    next0 = jnp.min(jnp.where(valid, flat, _PAD))
    _, cnt_fin = jax.lax.while_loop(cond, body, (next0, 0))

    def pad_cond(state):
        p, cnt = state
        return (p < _PRE) & (cnt < _POST)

    def pad_body(state):
        p, cnt = state
        emit(cnt, p, -1.0)
        dead = (keep_ref[...] == 0.0) & (flat > p) & (flat < _PRE)
        nxt = jnp.min(jnp.where(dead, flat, _PRE))
        return nxt, cnt + 1

    @pl.when(cnt_fin < _POST)
    def _():
        dead0 = (keep_ref[...] == 0.0) & (flat < _PRE)
        p0 = jnp.min(jnp.where(dead0, flat, _PRE))
        jax.lax.while_loop(pad_cond, pad_body, (p0, cnt_fin))


def kernel(boxes, scores):
    masked = jnp.where(scores >= _SCORE_THR, scores, -1.0)
    top_scores, idx = jax.lax.top_k(masked, _PRE)
    top_boxes = jnp.take(boxes, idx, axis=0)

    pad = _PAD - _PRE
    sflat = jnp.pad(top_scores, (0, pad), constant_values=-1.0)
    iflat = jnp.pad(idx, (0, pad))
    bp = jnp.pad(top_boxes, ((0, pad), (0, 0)))
    x1f = bp[:, 0]
    y1f = bp[:, 1]
    x2f = bp[:, 2]
    y2f = bp[:, 3]

    vm = pl.BlockSpec(memory_space=pltpu.MemorySpace.VMEM)
    sm = pl.BlockSpec(memory_space=pltpu.MemorySpace.SMEM)
    out_shapes = [jax.ShapeDtypeStruct((1, 128), jnp.float32)] * 5 + [
        jax.ShapeDtypeStruct((1, 128), jnp.int32)
    ]
    ox1, oy1, ox2, oy2, osc, oidx = pl.pallas_call(
        _nms_kernel,
        out_shape=out_shapes,
        in_specs=[vm] * 5 + [sm] * 6,
        out_specs=[vm] * 6,
        scratch_shapes=[
            pltpu.VMEM((_ROWS, 128), jnp.float32),
        ],
    )(
        x1f.reshape(_ROWS, 128), y1f.reshape(_ROWS, 128),
        x2f.reshape(_ROWS, 128), y2f.reshape(_ROWS, 128),
        sflat.reshape(_ROWS, 128),
        x1f, y1f, x2f, y2f, sflat, iflat,
    )

    sel_boxes = jnp.stack(
        [ox1[0, :_POST], oy1[0, :_POST], ox2[0, :_POST], oy2[0, :_POST]],
        axis=1,
    )
    return sel_boxes, osc[0, :_POST], oidx[0, :_POST]
